# baseline (speedup 1.0000x reference)
import functools

import jax
import jax.numpy as jnp
from jax import lax
from jax.experimental import pallas as pl
from jax.experimental.pallas import tpu as pltpu
from jax.experimental.pallas import tpu_sc as plsc

_NC = 2
_NS = 16
_NW = _NC * _NS


@functools.lru_cache(maxsize=None)
def _make_deg(E, N, C, NCH, NCHP, NK, SEG):
    mesh = plsc.VectorSubcoreMesh(core_axis_name="c", subcore_axis_name="s")

    @functools.partial(
        pl.kernel, mesh=mesh,
        out_type=jax.ShapeDtypeStruct((NK * _NW * SEG,), jnp.float32),
        compiler_params=pltpu.CompilerParams(needs_layout_passes=False),
        scratch_types=[
            pltpu.VMEM((NCHP, C), jnp.int32),
            pltpu.VMEM((N + 16,), jnp.float32),
        ],
    )
    def deg_k(dst_hbm, out_hbm, dst_v, hist):
        c = lax.axis_index("c")
        s = lax.axis_index("s")
        w = c * _NS + s

        def zero(i, _):
            hist[pl.ds(i * 16, 16)] = jnp.zeros((16,), jnp.float32)
            return 0
        lax.fori_loop(0, N // 16 + 1, zero, 0)

        pltpu.sync_copy(dst_hbm.at[pl.ds(w * NCHP, NCHP)], dst_v)
        ones = jnp.full((16,), 1.0, jnp.float32)

        def body(r, _):
            for k in range(C // 16):
                idx = dst_v[r, pl.ds(k * 16, 16)]
                plsc.addupdate_scatter(hist, [idx], ones)
            return 0
        lax.fori_loop(0, NCH, body, 0)

        for k in range(NK):
            pltpu.sync_copy(hist.at[pl.ds(k * SEG, SEG)],
                            out_hbm.at[pl.ds((k * _NW + w) * SEG, SEG)])

    return deg_k


@functools.lru_cache(maxsize=None)
def _make_agg(N, D, C, NCH, NCHP):
    mesh = plsc.VectorSubcoreMesh(core_axis_name="c", subcore_axis_name="s")
    NP = N + 8
    RPT = (N // (8 * _NS)) * 8
    TAIL = N - _NS * RPT
    NZ = RPT // C
    REM = RPT - NZ * C
    NPAIR = NCH // 2

    SB = 32
    PHASES = [(p * SB, min(SB, NCH - p * SB)) for p in range((NCH + SB - 1) // SB)]

    @functools.partial(
        pl.kernel, mesh=mesh,
        out_type=jax.ShapeDtypeStruct((_NC * N, D), jnp.float32),
        scratch_types=[
            pltpu.VMEM((SB, C), jnp.int32),
            pltpu.VMEM((NCHP, C), jnp.int32),
            pltpu.VMEM((C, D), jnp.float32),
            pltpu.VMEM((C, D), jnp.float32),
            pltpu.VMEM_SHARED((NP, D), jnp.float32),
            pltpu.SemaphoreType.DMA((2,)),
            pltpu.SemaphoreType.DMA((2,)),
        ],
    )
    def agg_k(hp_hbm, src_hbm, dst_hbm, out_hbm,
              sb_v, dst_v, rows0, rows1, acc, gsems, ssems):
        gsem0, gsem1 = gsems.at[0], gsems.at[1]
        ssem0, ssem1 = ssems.at[0], ssems.at[1]
        c = lax.axis_index("c")
        s = lax.axis_index("s")
        w = c * _NS + s

        def zrow(r, _):
            for j in range(D // 16):
                rows0[r, pl.ds(j * 16, 16)] = jnp.zeros((16,), jnp.float32)
            return 0
        lax.fori_loop(0, C, zrow, 0)
        for k in range(NZ):
            pltpu.sync_copy(rows0, acc.at[pl.ds(s * RPT + k * C, C)])
        if REM:
            pltpu.sync_copy(rows0.at[pl.ds(0, REM)],
                            acc.at[pl.ds(s * RPT + NZ * C, REM)])

        @pl.when(s == _NS - 1)
        def _():
            pltpu.sync_copy(rows0.at[pl.ds(0, TAIL)],
                            acc.at[pl.ds(_NS * RPT, TAIL)])
            pltpu.sync_copy(rows0.at[pl.ds(0, NP - N)], acc.at[pl.ds(N, NP - N)])

        pltpu.sync_copy(dst_hbm.at[pl.ds(w * NCHP, NCHP)], dst_v)
        plsc.subcore_barrier()

        def drain_gather(q, rows, sem):
            pltpu.make_async_copy(hp_hbm.at[sb_v.at[q]], rows, sem).wait()

        def drain_scatter(rows, sem):
            pltpu.make_async_copy(rows, acc.at[dst_v.at[0]], sem).wait()

        for jbase, cnt in PHASES:
            cnt_p = min(SB, NCHP - jbase)
            pltpu.sync_copy(src_hbm.at[pl.ds(w * NCHP + jbase, cnt_p)],
                            sb_v.at[pl.ds(0, cnt_p)])
            npair = cnt // 2
            pltpu.async_copy(hp_hbm.at[sb_v.at[0]], rows0, gsem0)
            pltpu.async_copy(hp_hbm.at[sb_v.at[1]], rows1, gsem1)

            def body(i, _, jbase=jbase, cnt=cnt):
                q0 = 2 * i
                drain_gather(q0, rows0, gsem0)
                pltpu.async_copy(rows0, acc.at[dst_v.at[jbase + q0]],
                                 ssem0, add=True)
                drain_gather(q0 + 1, rows1, gsem1)
                pltpu.async_copy(rows1, acc.at[dst_v.at[jbase + q0 + 1]],
                                 ssem1, add=True)
                drain_scatter(rows0, ssem0)

                @pl.when(q0 + 2 < cnt)
                def _():
                    pltpu.async_copy(hp_hbm.at[sb_v.at[q0 + 2]], rows0, gsem0)
                drain_scatter(rows1, ssem1)

                @pl.when(q0 + 3 < cnt)
                def _():
                    pltpu.async_copy(hp_hbm.at[sb_v.at[q0 + 3]], rows1, gsem1)
                return 0
            lax.fori_loop(0, npair, body, 0)

            if cnt % 2:
                drain_gather(cnt - 1, rows0, gsem0)
                pltpu.async_copy(rows0, acc.at[dst_v.at[jbase + cnt - 1]],
                                 ssem0, add=True)
                drain_scatter(rows0, ssem0)

        plsc.subcore_barrier()
        pltpu.sync_copy(acc.at[pl.ds(s * RPT, RPT)],
                        out_hbm.at[pl.ds(c * N + s * RPT, RPT)])

        @pl.when(s == _NS - 1)
        def _():
            pltpu.sync_copy(acc.at[pl.ds(_NS * RPT, TAIL)],
                            out_hbm.at[pl.ds(c * N + _NS * RPT, TAIL)])

    return agg_k


def _dinv_of(deg_ref):
    deg = jnp.sum(deg_ref[0], axis=0) + 2.0
    return lax.rsqrt(deg)[:, None]


def _tc1_body(deg_ref, x_ref, w_ref, hp_ref):
    h = jnp.dot(x_ref[...], w_ref[...], preferred_element_type=jnp.float32)
    hp_ref[...] = h * _dinv_of(deg_ref)


def _tc2_body(deg_ref, p0_ref, p1_ref, hp_ref, b_ref, w_ref, o_ref):
    dinv = _dinv_of(deg_ref)
    agg = p0_ref[...] + p1_ref[...] + 2.0 * hp_ref[...]
    h = jnp.maximum(agg * dinv + b_ref[...], 0.0)
    o_ref[...] = dinv * jnp.dot(h, w_ref[...],
                                preferred_element_type=jnp.float32)


def _make_tc3_body(G, NB, R):
    def _tc3_body(deg_ref, p0_ref, p1_ref, hp_ref, b_ref, batch_ref,
                  wf_ref, bf_ref, o_ref, pooled):
        i = pl.program_id(0)
        dinv = _dinv_of(deg_ref)
        agg = p0_ref[...] + p1_ref[...] + 2.0 * hp_ref[...]
        h2 = jnp.maximum(agg * dinv + b_ref[...], 0.0)

        @pl.when(i == 0)
        def _():
            pooled[...] = jnp.full(pooled.shape, -jnp.inf, jnp.float32)

        b = batch_ref[0]
        g0 = batch_ref[0, 0, 0]
        g1 = batch_ref[0, R - 1, 0]

        def seg(g, _):
            mask = (b == g)
            m = jnp.max(jnp.where(mask, h2, -jnp.inf), axis=0, keepdims=True)
            cur = pooled[pl.ds(g, 1), :]
            pooled[pl.ds(g, 1), :] = jnp.maximum(cur, m)
            return 0
        lax.fori_loop(g0, g1 + 1, seg, 0)

        @pl.when(i == NB - 1)
        def _():
            o_ref[...] = jnp.dot(pooled[...], wf_ref[...],
                                 preferred_element_type=jnp.float32) + bf_ref[...]
    return _tc3_body


def kernel(x, edge_index, batch, W1, b1, W2, b2, Wf, bf):
    N, D = x.shape
    E = edge_index.shape[1]
    G = 64
    OUT = Wf.shape[1]
    C = 128
    EPW = E // _NW
    NCH = -(-EPW // C)
    NCHP = (NCH + 7) // 8 * 8
    R = 1000
    NB = N // R

    def _pad3(a, fill):
        a2 = a.reshape(_NW, EPW)
        a2 = jnp.pad(a2, ((0, 0), (0, NCHP * C - EPW)), constant_values=fill)
        return a2.reshape(_NW * NCHP, C)

    src2 = _pad3(edge_index[0], 0)
    dst2 = _pad3(edge_index[1], N)

    deg_flat = _make_deg(E, N, C, NCH, NCHP, NB, R)(dst2)
    deg3 = deg_flat.reshape(NB, _NW, R)

    deg_spec = pl.BlockSpec((1, _NW, R), lambda i: (i, 0, 0))
    row_spec = pl.BlockSpec((R, D), lambda i: (i, 0))
    p0_spec = pl.BlockSpec((R, D), lambda i: (i, 0))
    p1_spec = pl.BlockSpec((R, D), lambda i, _nb=NB: (i + _nb, 0))
    w_spec = pl.BlockSpec((D, D), lambda i: (0, 0))
    b_spec = pl.BlockSpec((1, D), lambda i: (0, 0))

    hp1 = pl.pallas_call(
        _tc1_body,
        grid=(NB,),
        in_specs=[deg_spec, row_spec, w_spec],
        out_specs=row_spec,
        out_shape=jax.ShapeDtypeStruct((N, D), jnp.float32),
    )(deg3, x, W1)

    agg = _make_agg(N, D, C, NCH, NCHP)
    parts1 = agg(hp1, src2, dst2)

    hp2 = pl.pallas_call(
        _tc2_body,
        grid=(NB,),
        in_specs=[deg_spec, p0_spec, p1_spec, row_spec, b_spec, w_spec],
        out_specs=row_spec,
        out_shape=jax.ShapeDtypeStruct((N, D), jnp.float32),
    )(deg3, parts1, parts1, hp1, b1.reshape(1, D), W2)

    parts2 = agg(hp2, src2, dst2)

    out = pl.pallas_call(
        _make_tc3_body(G, NB, R),
        grid=(NB,),
        in_specs=[deg_spec, p0_spec, p1_spec, row_spec, b_spec,
                  pl.BlockSpec((1, R, 1), lambda i: (i, 0, 0)),
                  pl.BlockSpec((D, OUT), lambda i: (0, 0)),
                  pl.BlockSpec((1, OUT), lambda i: (0, 0))],
        out_specs=pl.BlockSpec((G, OUT), lambda i: (0, 0)),
        out_shape=jax.ShapeDtypeStruct((G, OUT), jnp.float32),
        scratch_shapes=[pltpu.VMEM((G, D), jnp.float32)],
    )(deg3, parts2, parts2, hp2, b2.reshape(1, D),
      batch.reshape(NB, R, 1), Wf, bf.reshape(1, OUT))

    return out


# speedup vs baseline: 1.2470x; 1.2470x over previous
import functools

import jax
import jax.numpy as jnp
from jax import lax
from jax.experimental import pallas as pl
from jax.experimental.pallas import tpu as pltpu
from jax.experimental.pallas import tpu_sc as plsc

_NC = 2
_NS = 16
_NW = _NC * _NS


@functools.lru_cache(maxsize=None)
def _make_deg(E, N, C, NCH, NCHP, NK, SEG):
    mesh = plsc.VectorSubcoreMesh(core_axis_name="c", subcore_axis_name="s")

    @functools.partial(
        pl.kernel, mesh=mesh,
        out_type=jax.ShapeDtypeStruct((NK * _NW * SEG,), jnp.float32),
        compiler_params=pltpu.CompilerParams(needs_layout_passes=False),
        scratch_types=[
            pltpu.VMEM((NCHP, C), jnp.int32),
            pltpu.VMEM((N + 16,), jnp.float32),
        ],
    )
    def deg_k(dst_hbm, out_hbm, dst_v, hist):
        c = lax.axis_index("c")
        s = lax.axis_index("s")
        w = c * _NS + s

        def zero(i, _):
            hist[pl.ds(i * 16, 16)] = jnp.zeros((16,), jnp.float32)
            return 0
        lax.fori_loop(0, N // 16 + 1, zero, 0)

        pltpu.sync_copy(dst_hbm.at[pl.ds(w * NCHP, NCHP)], dst_v)
        ones = jnp.full((16,), 1.0, jnp.float32)

        def body(r, _):
            for k in range(C // 16):
                idx = dst_v[r, pl.ds(k * 16, 16)]
                plsc.addupdate_scatter(hist, [idx], ones)
            return 0
        lax.fori_loop(0, NCH, body, 0)

        for k in range(NK):
            pltpu.sync_copy(hist.at[pl.ds(k * SEG, SEG)],
                            out_hbm.at[pl.ds((k * _NW + w) * SEG, SEG)])

    return deg_k


@functools.lru_cache(maxsize=None)
def _make_agg(N, D, C, NCH, NCHP):
    mesh = plsc.VectorSubcoreMesh(core_axis_name="c", subcore_axis_name="s")
    NP = N + 8
    RPT = (N // (8 * _NS)) * 8
    TAIL = N - _NS * RPT
    NZ = RPT // C
    REM = RPT - NZ * C
    NPAIR = NCH // 2

    SRC_R = NCHP * C // 128

    @functools.partial(
        pl.kernel, mesh=mesh,
        out_type=jax.ShapeDtypeStruct((_NC * N, D), jnp.float32),
        scratch_types=[
            pltpu.VMEM((SRC_R, 128), jnp.int32),
            pltpu.VMEM((NCHP, C), jnp.int32),
            pltpu.VMEM((C, D), jnp.float32),
            pltpu.VMEM((C, D), jnp.float32),
            pltpu.VMEM_SHARED((NP, D), jnp.float32),
            pltpu.SemaphoreType.DMA((2,)),
            pltpu.SemaphoreType.DMA((2,)),
        ],
    )
    def agg_k(hp_hbm, src_hbm, dst_hbm, out_hbm,
              src_v, dst_v, rows0, rows1, acc, gsems, ssems):
        gsem0, gsem1 = gsems.at[0], gsems.at[1]
        ssem0, ssem1 = ssems.at[0], ssems.at[1]
        c = lax.axis_index("c")
        s = lax.axis_index("s")
        w = c * _NS + s

        def zrow(r, _):
            for j in range(D // 16):
                rows0[r, pl.ds(j * 16, 16)] = jnp.zeros((16,), jnp.float32)
            return 0
        lax.fori_loop(0, C, zrow, 0)
        for k in range(NZ):
            pltpu.sync_copy(rows0, acc.at[pl.ds(s * RPT + k * C, C)])
        if REM:
            pltpu.sync_copy(rows0.at[pl.ds(0, REM)],
                            acc.at[pl.ds(s * RPT + NZ * C, REM)])

        @pl.when(s == _NS - 1)
        def _():
            pltpu.sync_copy(rows0.at[pl.ds(0, TAIL)],
                            acc.at[pl.ds(_NS * RPT, TAIL)])
            pltpu.sync_copy(rows0.at[pl.ds(0, NP - N)], acc.at[pl.ds(N, NP - N)])

        pltpu.sync_copy(src_hbm.at[pl.ds(w * SRC_R, SRC_R)], src_v)
        pltpu.sync_copy(dst_hbm.at[pl.ds(w * NCHP, NCHP)], dst_v)
        plsc.subcore_barrier()

        def sidx(i, half):
            return src_v.at[i, pl.ds(half * C, C)]

        def drain_scatter(rows, sem):
            pltpu.make_async_copy(rows, acc.at[dst_v.at[0]], sem).wait()

        pltpu.async_copy(hp_hbm.at[sidx(0, 0)], rows0, gsem0)

        def pair(i, _):
            j0 = 2 * i
            pltpu.async_copy(hp_hbm.at[sidx(i, 1)], rows1, gsem1)
            pltpu.make_async_copy(hp_hbm.at[sidx(i, 0)], rows0, gsem0).wait()
            pltpu.async_copy(rows0, acc.at[dst_v.at[j0]], ssem0, add=True)
            pltpu.make_async_copy(hp_hbm.at[sidx(i, 1)], rows1, gsem1).wait()
            pltpu.async_copy(rows1, acc.at[dst_v.at[j0 + 1]], ssem1, add=True)
            drain_scatter(rows0, ssem0)
            pltpu.async_copy(hp_hbm.at[sidx(i + 1, 0)], rows0, gsem0)
            drain_scatter(rows1, ssem1)
            return 0
        lax.fori_loop(0, NPAIR, pair, 0)

        if NCH % 2:
            pltpu.make_async_copy(hp_hbm.at[sidx(NPAIR, 0)], rows0, gsem0).wait()
            pltpu.sync_copy(rows0, acc.at[dst_v.at[NCH - 1]], add=True)

        plsc.subcore_barrier()
        pltpu.sync_copy(acc.at[pl.ds(s * RPT, RPT)],
                        out_hbm.at[pl.ds(c * N + s * RPT, RPT)])

        @pl.when(s == _NS - 1)
        def _():
            pltpu.sync_copy(acc.at[pl.ds(_NS * RPT, TAIL)],
                            out_hbm.at[pl.ds(c * N + _NS * RPT, TAIL)])

    return agg_k


def _dinv_of(deg_ref):
    deg = jnp.sum(deg_ref[0], axis=0) + 2.0
    return lax.rsqrt(deg)[:, None]


def _tc1_body(deg_ref, x_ref, w_ref, hp_ref):
    h = jnp.dot(x_ref[...], w_ref[...], preferred_element_type=jnp.float32)
    hp_ref[...] = h * _dinv_of(deg_ref)


def _tc2_body(deg_ref, p0_ref, p1_ref, hp_ref, b_ref, w_ref, o_ref):
    dinv = _dinv_of(deg_ref)
    agg = p0_ref[...] + p1_ref[...] + 2.0 * hp_ref[...]
    h = jnp.maximum(agg * dinv + b_ref[...], 0.0)
    o_ref[...] = dinv * jnp.dot(h, w_ref[...],
                                preferred_element_type=jnp.float32)


def _make_tc3_body(G, NB, R):
    def _tc3_body(deg_ref, p0_ref, p1_ref, hp_ref, b_ref, batch_ref,
                  wf_ref, bf_ref, o_ref, pooled):
        i = pl.program_id(0)
        dinv = _dinv_of(deg_ref)
        agg = p0_ref[...] + p1_ref[...] + 2.0 * hp_ref[...]
        h2 = jnp.maximum(agg * dinv + b_ref[...], 0.0)

        @pl.when(i == 0)
        def _():
            pooled[...] = jnp.full(pooled.shape, -jnp.inf, jnp.float32)

        b = batch_ref[0]
        g0 = batch_ref[0, 0, 0]
        g1 = batch_ref[0, R - 1, 0]

        def seg(g, _):
            mask = (b == g)
            m = jnp.max(jnp.where(mask, h2, -jnp.inf), axis=0, keepdims=True)
            cur = pooled[pl.ds(g, 1), :]
            pooled[pl.ds(g, 1), :] = jnp.maximum(cur, m)
            return 0
        lax.fori_loop(g0, g1 + 1, seg, 0)

        @pl.when(i == NB - 1)
        def _():
            o_ref[...] = jnp.dot(pooled[...], wf_ref[...],
                                 preferred_element_type=jnp.float32) + bf_ref[...]
    return _tc3_body


def kernel(x, edge_index, batch, W1, b1, W2, b2, Wf, bf):
    N, D = x.shape
    E = edge_index.shape[1]
    G = 64
    OUT = Wf.shape[1]
    C = 64
    EPW = E // _NW
    NCH = -(-EPW // C)
    assert NCH % 2 == 1
    EPP = NCH * C
    NCHP = (NCH + 7) // 8 * 8
    R = 1000
    NB = N // R

    d2 = edge_index[1].reshape(_NW, EPW)
    d2 = jnp.pad(d2, ((0, 0), (0, EPP - EPW)), constant_values=N)
    d3 = d2.reshape(_NW, NCH, C)
    d3 = jnp.pad(d3, ((0, 0), (0, NCHP - NCH), (0, 0)))
    dst2 = d3.reshape(_NW * NCHP, C)

    s2 = edge_index[0].reshape(_NW, EPW)
    s2 = jnp.pad(s2, ((0, 0), (0, NCHP * C - EPW)))
    src2 = s2.reshape(_NW * (NCHP * C // 128), 128)

    deg_flat = _make_deg(E, N, C, NCH, NCHP, NB, R)(dst2)
    deg3 = deg_flat.reshape(NB, _NW, R)

    deg_spec = pl.BlockSpec((1, _NW, R), lambda i: (i, 0, 0))
    row_spec = pl.BlockSpec((R, D), lambda i: (i, 0))
    p0_spec = pl.BlockSpec((R, D), lambda i: (i, 0))
    p1_spec = pl.BlockSpec((R, D), lambda i, _nb=NB: (i + _nb, 0))
    w_spec = pl.BlockSpec((D, D), lambda i: (0, 0))
    b_spec = pl.BlockSpec((1, D), lambda i: (0, 0))

    hp1 = pl.pallas_call(
        _tc1_body,
        grid=(NB,),
        in_specs=[deg_spec, row_spec, w_spec],
        out_specs=row_spec,
        out_shape=jax.ShapeDtypeStruct((N, D), jnp.float32),
    )(deg3, x, W1)

    agg = _make_agg(N, D, C, NCH, NCHP)
    parts1 = agg(hp1, src2, dst2)

    hp2 = pl.pallas_call(
        _tc2_body,
        grid=(NB,),
        in_specs=[deg_spec, p0_spec, p1_spec, row_spec, b_spec, w_spec],
        out_specs=row_spec,
        out_shape=jax.ShapeDtypeStruct((N, D), jnp.float32),
    )(deg3, parts1, parts1, hp1, b1.reshape(1, D), W2)

    parts2 = agg(hp2, src2, dst2)

    out = pl.pallas_call(
        _make_tc3_body(G, NB, R),
        grid=(NB,),
        in_specs=[deg_spec, p0_spec, p1_spec, row_spec, b_spec,
                  pl.BlockSpec((1, R, 1), lambda i: (i, 0, 0)),
                  pl.BlockSpec((D, OUT), lambda i: (0, 0)),
                  pl.BlockSpec((1, OUT), lambda i: (0, 0))],
        out_specs=pl.BlockSpec((G, OUT), lambda i: (0, 0)),
        out_shape=jax.ShapeDtypeStruct((G, OUT), jnp.float32),
        scratch_shapes=[pltpu.VMEM((G, D), jnp.float32)],
    )(deg3, parts2, parts2, hp2, b2.reshape(1, D),
      batch.reshape(NB, R, 1), Wf, bf.reshape(1, OUT))

    return out


# speedup vs baseline: 1.9593x; 1.5711x over previous
import functools

import jax
import jax.numpy as jnp
from jax import lax
from jax.experimental import pallas as pl
from jax.experimental.pallas import tpu as pltpu
from jax.experimental.pallas import tpu_sc as plsc

_NC = 2
_NS = 16
_NW = _NC * _NS


@functools.lru_cache(maxsize=None)
def _make_deg(E, N, C, NCH, NCHP, NK, SEG):
    mesh = plsc.VectorSubcoreMesh(core_axis_name="c", subcore_axis_name="s")

    @functools.partial(
        pl.kernel, mesh=mesh,
        out_type=jax.ShapeDtypeStruct((NK * _NW * SEG,), jnp.float32),
        compiler_params=pltpu.CompilerParams(needs_layout_passes=False),
        scratch_types=[
            pltpu.VMEM((NCHP, C), jnp.int32),
            pltpu.VMEM((N + 16,), jnp.float32),
        ],
    )
    def deg_k(dst_hbm, out_hbm, dst_v, hist):
        c = lax.axis_index("c")
        s = lax.axis_index("s")
        w = c * _NS + s

        def zero(i, _):
            hist[pl.ds(i * 16, 16)] = jnp.zeros((16,), jnp.float32)
            return 0
        lax.fori_loop(0, N // 16 + 1, zero, 0)

        pltpu.sync_copy(dst_hbm.at[pl.ds(w * NCHP, NCHP)], dst_v)
        ones = jnp.full((16,), 1.0, jnp.float32)

        def body(r, _):
            for k in range(C // 16):
                idx = dst_v[r, pl.ds(k * 16, 16)]
                plsc.addupdate_scatter(hist, [idx], ones)
            return 0
        lax.fori_loop(0, NCH, body, 0)

        for k in range(NK):
            pltpu.sync_copy(hist.at[pl.ds(k * SEG, SEG)],
                            out_hbm.at[pl.ds((k * _NW + w) * SEG, SEG)])

    return deg_k


@functools.lru_cache(maxsize=None)
def _make_agg(N, D, C, NCH, NCHP):
    mesh = plsc.VectorSubcoreMesh(core_axis_name="c", subcore_axis_name="s")
    NP = N + 8
    RPT = (N // (8 * _NS)) * 8
    TAIL = N - _NS * RPT
    NZ = RPT // C
    REM = RPT - NZ * C
    NPAIR = NCH // 2

    SRC_W = NCHP * C

    @functools.partial(
        pl.kernel, mesh=mesh,
        out_type=jax.ShapeDtypeStruct((_NC * N, D), jnp.float32),
        scratch_types=[
            pltpu.VMEM((SRC_W,), jnp.int32),
            pltpu.VMEM((NCHP, C), jnp.int32),
            pltpu.VMEM((C, D), jnp.float32),
            pltpu.VMEM((C, D), jnp.float32),
            pltpu.VMEM_SHARED((NP, D), jnp.float32),
            pltpu.SemaphoreType.DMA((2,)),
        ],
    )
    def agg_k(hp_hbm, src_hbm, dst_hbm, out_hbm,
              src_v, dst_v, rows0, rows1, acc, gsems):
        gsem0, gsem1 = gsems.at[0], gsems.at[1]
        c = lax.axis_index("c")
        s = lax.axis_index("s")
        w = c * _NS + s

        def zrow(r, _):
            for j in range(D // 16):
                rows0[r, pl.ds(j * 16, 16)] = jnp.zeros((16,), jnp.float32)
            return 0
        lax.fori_loop(0, C, zrow, 0)
        for k in range(NZ):
            pltpu.sync_copy(rows0, acc.at[pl.ds(s * RPT + k * C, C)])
        if REM:
            pltpu.sync_copy(rows0.at[pl.ds(0, REM)],
                            acc.at[pl.ds(s * RPT + NZ * C, REM)])

        @pl.when(s == _NS - 1)
        def _():
            pltpu.sync_copy(rows0.at[pl.ds(0, TAIL)],
                            acc.at[pl.ds(_NS * RPT, TAIL)])
            pltpu.sync_copy(rows0.at[pl.ds(0, NP - N)], acc.at[pl.ds(N, NP - N)])

        pltpu.sync_copy(src_hbm.at[pl.ds(w * SRC_W, SRC_W)], src_v)
        pltpu.sync_copy(dst_hbm.at[pl.ds(w * NCHP, NCHP)], dst_v)
        plsc.subcore_barrier()

        def sidx(j):
            return src_v.at[pl.ds(j * C, C)]

        pltpu.async_copy(hp_hbm.at[sidx(0)], rows0, gsem0)

        def pair(i, _):
            j0 = 2 * i
            pltpu.async_copy(hp_hbm.at[sidx(j0 + 1)], rows1, gsem1)
            pltpu.make_async_copy(hp_hbm.at[sidx(j0)], rows0, gsem0).wait()
            pltpu.sync_copy(rows0, acc.at[dst_v.at[j0]], add=True)
            pltpu.async_copy(hp_hbm.at[sidx(j0 + 2)], rows0, gsem0)
            pltpu.make_async_copy(hp_hbm.at[sidx(j0 + 1)], rows1, gsem1).wait()
            pltpu.sync_copy(rows1, acc.at[dst_v.at[j0 + 1]], add=True)
            return 0
        lax.fori_loop(0, NPAIR, pair, 0)

        if NCH % 2:
            pltpu.make_async_copy(hp_hbm.at[sidx(NCH - 1)], rows0, gsem0).wait()
            pltpu.sync_copy(rows0, acc.at[dst_v.at[NCH - 1]], add=True)

        plsc.subcore_barrier()
        pltpu.sync_copy(acc.at[pl.ds(s * RPT, RPT)],
                        out_hbm.at[pl.ds(c * N + s * RPT, RPT)])

        @pl.when(s == _NS - 1)
        def _():
            pltpu.sync_copy(acc.at[pl.ds(_NS * RPT, TAIL)],
                            out_hbm.at[pl.ds(c * N + _NS * RPT, TAIL)])

    return agg_k


def _dinv_of(deg_ref):
    deg = jnp.sum(deg_ref[0], axis=0) + 2.0
    return lax.rsqrt(deg)[:, None]


def _tc1_body(deg_ref, x_ref, w_ref, hp_ref):
    h = jnp.dot(x_ref[...], w_ref[...], preferred_element_type=jnp.float32)
    hp_ref[...] = h * _dinv_of(deg_ref)


def _tc2_body(deg_ref, p0_ref, p1_ref, hp_ref, b_ref, w_ref, o_ref):
    dinv = _dinv_of(deg_ref)
    agg = p0_ref[...] + p1_ref[...] + 2.0 * hp_ref[...]
    h = jnp.maximum(agg * dinv + b_ref[...], 0.0)
    o_ref[...] = dinv * jnp.dot(h, w_ref[...],
                                preferred_element_type=jnp.float32)


def _make_tc3_body(G, NB, R):
    def _tc3_body(deg_ref, p0_ref, p1_ref, hp_ref, b_ref, batch_ref,
                  wf_ref, bf_ref, o_ref, pooled):
        i = pl.program_id(0)
        dinv = _dinv_of(deg_ref)
        agg = p0_ref[...] + p1_ref[...] + 2.0 * hp_ref[...]
        h2 = jnp.maximum(agg * dinv + b_ref[...], 0.0)

        @pl.when(i == 0)
        def _():
            pooled[...] = jnp.full(pooled.shape, -jnp.inf, jnp.float32)

        b = batch_ref[0]
        g0 = batch_ref[0, 0, 0]
        g1 = batch_ref[0, R - 1, 0]

        def seg(g, _):
            mask = (b == g)
            m = jnp.max(jnp.where(mask, h2, -jnp.inf), axis=0, keepdims=True)
            cur = pooled[pl.ds(g, 1), :]
            pooled[pl.ds(g, 1), :] = jnp.maximum(cur, m)
            return 0
        lax.fori_loop(g0, g1 + 1, seg, 0)

        @pl.when(i == NB - 1)
        def _():
            o_ref[...] = jnp.dot(pooled[...], wf_ref[...],
                                 preferred_element_type=jnp.float32) + bf_ref[...]
    return _tc3_body


def kernel(x, edge_index, batch, W1, b1, W2, b2, Wf, bf):
    N, D = x.shape
    E = edge_index.shape[1]
    G = 64
    OUT = Wf.shape[1]
    C = 80
    EPW = E // _NW
    NCH = -(-EPW // C)
    assert NCH % 2 == 1
    EPP = NCH * C
    NCHP = (NCH + 7) // 8 * 8
    R = 1000
    NB = N // R

    d2 = edge_index[1].reshape(_NW, EPW)
    d2 = jnp.pad(d2, ((0, 0), (0, EPP - EPW)), constant_values=N)
    d3 = d2.reshape(_NW, NCH, C)
    d3 = jnp.pad(d3, ((0, 0), (0, NCHP - NCH), (0, 0)))
    dst2 = d3.reshape(_NW * NCHP, C)

    s2 = edge_index[0].reshape(_NW, EPW)
    s2 = jnp.pad(s2, ((0, 0), (0, NCHP * C - EPW)))
    src2 = s2.reshape(_NW * NCHP * C)

    deg_flat = _make_deg(E, N, C, NCH, NCHP, NB, R)(dst2)
    deg3 = deg_flat.reshape(NB, _NW, R)

    deg_spec = pl.BlockSpec((1, _NW, R), lambda i: (i, 0, 0))
    row_spec = pl.BlockSpec((R, D), lambda i: (i, 0))
    p0_spec = pl.BlockSpec((R, D), lambda i: (i, 0))
    p1_spec = pl.BlockSpec((R, D), lambda i, _nb=NB: (i + _nb, 0))
    w_spec = pl.BlockSpec((D, D), lambda i: (0, 0))
    b_spec = pl.BlockSpec((1, D), lambda i: (0, 0))

    hp1 = pl.pallas_call(
        _tc1_body,
        grid=(NB,),
        in_specs=[deg_spec, row_spec, w_spec],
        out_specs=row_spec,
        out_shape=jax.ShapeDtypeStruct((N, D), jnp.float32),
    )(deg3, x, W1)

    agg = _make_agg(N, D, C, NCH, NCHP)
    parts1 = agg(hp1, src2, dst2)

    hp2 = pl.pallas_call(
        _tc2_body,
        grid=(NB,),
        in_specs=[deg_spec, p0_spec, p1_spec, row_spec, b_spec, w_spec],
        out_specs=row_spec,
        out_shape=jax.ShapeDtypeStruct((N, D), jnp.float32),
    )(deg3, parts1, parts1, hp1, b1.reshape(1, D), W2)

    parts2 = agg(hp2, src2, dst2)

    out = pl.pallas_call(
        _make_tc3_body(G, NB, R),
        grid=(NB,),
        in_specs=[deg_spec, p0_spec, p1_spec, row_spec, b_spec,
                  pl.BlockSpec((1, R, 1), lambda i: (i, 0, 0)),
                  pl.BlockSpec((D, OUT), lambda i: (0, 0)),
                  pl.BlockSpec((1, OUT), lambda i: (0, 0))],
        out_specs=pl.BlockSpec((G, OUT), lambda i: (0, 0)),
        out_shape=jax.ShapeDtypeStruct((G, OUT), jnp.float32),
        scratch_shapes=[pltpu.VMEM((G, D), jnp.float32)],
    )(deg3, parts2, parts2, hp2, b2.reshape(1, D),
      batch.reshape(NB, R, 1), Wf, bf.reshape(1, OUT))

    return out
